# baseline (speedup 1.0000x reference)
import functools

import jax
import jax.numpy as jnp
from jax import lax
from jax.experimental import pallas as pl
from jax.experimental.pallas import tpu as pltpu
from jax.experimental.pallas import tpu_sc as plsc

BATCH = 4096
WIDTH = 5000
N_ATTRIBUTES = 50
N_VALUES = 100
CURRICULUM_LEVEL = 25
HALF = 2560
LANES = 16
CHUNK = 8
ROWS = 512
SC_ROWS = 1024
TC_ROWS = BATCH - SC_ROWS


def _threefry_mantissa(counts_lo):
    u32 = jnp.uint32
    ks0 = u32(0)
    ks1 = u32(42)
    ks2 = u32(0x1BD11BDA ^ 42)

    def rotl(x, r):
        return lax.shift_left(x, u32(r)) | lax.shift_right_logical(
            x, u32(32 - r))

    def four_rounds(x0, x1, rots):
        for r in rots:
            x0 = x0 + x1
            x1 = rotl(x1, r) ^ x0
        return x0, x1

    R0 = (13, 15, 26, 6)
    R1 = (17, 29, 16, 24)
    x0 = jnp.zeros_like(counts_lo) + ks0
    x1 = counts_lo + ks1
    x0, x1 = four_rounds(x0, x1, R0)
    x0 = x0 + ks1
    x1 = x1 + ks2 + u32(1)
    x0, x1 = four_rounds(x0, x1, R1)
    x0 = x0 + ks2
    x1 = x1 + ks0 + u32(2)
    x0, x1 = four_rounds(x0, x1, R0)
    x0 = x0 + ks0
    x1 = x1 + ks1 + u32(3)
    x0, x1 = four_rounds(x0, x1, R1)
    x0 = x0 + ks1
    x1 = x1 + ks2 + u32(4)
    x0, x1 = four_rounds(x0, x1, R0)
    x0 = x0 + ks2
    x1 = x1 + ks0 + u32(5)
    return lax.shift_right_logical(x0 ^ x1, u32(9))


def _n_revealed_lanes(row0):
    rows25 = (row0 + lax.iota(jnp.int32, LANES)) * CURRICULUM_LEVEL

    def step(_, carry):
        best_m, best_j, jv = carry
        counts = (rows25 + jv).astype(jnp.uint32)
        m = _threefry_mantissa(counts).astype(jnp.int32)
        take = m > best_m
        return (jnp.where(take, m, best_m), jnp.where(take, jv, best_j),
                jv + 1)

    init = (jnp.full((LANES,), -1, jnp.int32), jnp.zeros((LANES,), jnp.int32),
            jnp.zeros((LANES,), jnp.int32))
    best_m, best_j, _ = lax.fori_loop(0, CURRICULUM_LEVEL, step, init)
    return best_j + 1


def _zero_right_tail(buf, zeros16):
    ntail8 = (WIDTH - HALF) // 128
    for r in range(CHUNK):
        def zinit(t, _, r=r):
            s0 = HALF + t * 128
            for k in range(8):
                buf[r, pl.ds(s0 + k * 16, LANES)] = zeros16
            return 0
        lax.fori_loop(0, ntail8, zinit, 0)
        buf[r, pl.ds(WIDTH - LANES, LANES)] = zeros16


def _sc_body(x_hbm, masked_hbm, mask_hbm, buf0, buf1, mb0, mb1, sem0, sem1):
    info = plsc.get_sparse_core_info()
    nc = info.num_cores
    wid = lax.axis_index("s") * nc + lax.axis_index("c")
    rows_per_worker = SC_ROWS // (nc * info.num_subcores)
    nchunks = rows_per_worker // CHUNK
    base = wid * rows_per_worker
    zeros16 = jnp.zeros((LANES,), jnp.float32)
    lane = lax.iota(jnp.int32, LANES)
    bufs = (buf0, buf1)
    mbs = (mb0, mb1)
    sems = (sem0, sem1)

    def fill_in(c, b):
        r0 = base + c * CHUNK
        pltpu.sync_copy(x_hbm.at[pl.ds(TC_ROWS + r0, CHUNK), pl.ds(0, HALF)],
                        bufs[b].at[:, pl.ds(0, HALF)])

    def drain_out(c, b):
        r0 = base + c * CHUNK
        pltpu.make_async_copy(
            bufs[b], masked_hbm.at[pl.ds(r0, CHUNK), :], sems[b]).wait()
        pltpu.make_async_copy(
            mbs[b], mask_hbm.at[pl.ds(r0, CHUNK), :], sems[b]).wait()

    def compute(c, b):
        buf = bufs[b]
        maskbuf = mbs[b]
        r0 = base + c * CHUNK
        n_rev = _n_revealed_lanes(TC_ROWS + r0)
        for r in range(CHUNK):
            nr = n_rev[r]
            for off in (0, 16, 32, 34):
                mval = jnp.where(lane + jnp.int32(off) < nr,
                                 jnp.float32(1.0), zeros16)
                maskbuf[r, pl.ds(off, LANES)] = mval
            p = nr * N_VALUES
            b0 = (p // LANES) * LANES
            v = buf[r, pl.ds(b0, LANES)]
            buf[r, pl.ds(b0, LANES)] = jnp.where(b0 + lane < p, v, zeros16)
            nz8 = (HALF - b0 - LANES + 127) // 128

            def ztail(t, _, r=r, b0=b0):
                s0 = b0 + LANES + t * 128
                for k in range(8):
                    buf[r, pl.ds(s0 + k * 16, LANES)] = zeros16
                return 0
            lax.fori_loop(0, nz8, ztail, 0)

    def start_out(c, b):
        r0 = base + c * CHUNK
        pltpu.async_copy(bufs[b], masked_hbm.at[pl.ds(r0, CHUNK), :], sems[b])
        pltpu.async_copy(mbs[b], mask_hbm.at[pl.ds(r0, CHUNK), :], sems[b])

    fill_in(0, 0)
    fill_in(1, 1)
    _zero_right_tail(buf0, zeros16)
    _zero_right_tail(buf1, zeros16)
    for b in (0, 1):
        compute(b, b)
        start_out(b, b)

    def lap(step, _):
        for b in (0, 1):
            c = step * 2 + b
            drain_out(c - 2, b)
            fill_in(c, b)
            compute(c, b)
            start_out(c, b)
        return 0

    lax.fori_loop(1, nchunks // 2, lap, 0)
    for b in (0, 1):
        drain_out(nchunks - 2 + b, b)


def _sc_run(sender_input):
    mesh = plsc.VectorSubcoreMesh(core_axis_name="c", subcore_axis_name="s")
    run = functools.partial(
        pl.kernel,
        out_type=[
            jax.ShapeDtypeStruct((SC_ROWS, WIDTH), jnp.float32),
            jax.ShapeDtypeStruct((SC_ROWS, N_ATTRIBUTES), jnp.float32),
        ],
        mesh=mesh,
        scratch_types=[
            pltpu.VMEM((CHUNK, WIDTH), jnp.float32),
            pltpu.VMEM((CHUNK, WIDTH), jnp.float32),
            pltpu.VMEM((CHUNK, N_ATTRIBUTES), jnp.float32),
            pltpu.VMEM((CHUNK, N_ATTRIBUTES), jnp.float32),
            pltpu.SemaphoreType.DMA,
            pltpu.SemaphoreType.DMA,
        ],
    )(_sc_body)
    return run(sender_input)


def _n_revealed_block(base_row, rows):
    cl = CURRICULUM_LEVEL
    r = lax.broadcasted_iota(jnp.uint32, (rows, cl), 0)
    j = lax.broadcasted_iota(jnp.uint32, (rows, cl), 1)
    counts = (base_row.astype(jnp.uint32) + r) * jnp.uint32(cl) + j
    m = _threefry_mantissa(counts).astype(jnp.int32)
    row_max = jnp.max(m, axis=1, keepdims=True)
    ji = lax.broadcasted_iota(jnp.int32, (rows, cl), 1)
    win = jnp.min(jnp.where(m == row_max, ji, jnp.int32(cl)), axis=1,
                  keepdims=True)
    return win + 1


def _tc_body(x_ref, masked_ref, mask_ref):
    i = pl.program_id(0)
    n_rev = _n_revealed_block(i * ROWS, ROWS)
    a = lax.broadcasted_iota(jnp.int32, (ROWS, N_ATTRIBUTES), 1)
    mask_ref[...] = (a < n_rev).astype(jnp.float32)
    c = lax.broadcasted_iota(jnp.int32, (ROWS, HALF), 1)
    masked_ref[:, 0:HALF] = jnp.where(c < n_rev * N_VALUES, x_ref[...], 0.0)
    masked_ref[:, HALF:] = jnp.zeros((ROWS, WIDTH - HALF), jnp.float32)


def _tc_run(sender_input):
    grid = TC_ROWS // ROWS
    return pl.pallas_call(
        _tc_body,
        grid=(grid,),
        in_specs=[pl.BlockSpec((ROWS, HALF), lambda i: (i, 0))],
        out_specs=[
            pl.BlockSpec((ROWS, WIDTH), lambda i: (i, 0)),
            pl.BlockSpec((ROWS, N_ATTRIBUTES), lambda i: (i, 0)),
        ],
        out_shape=[
            jax.ShapeDtypeStruct((TC_ROWS, WIDTH), jnp.float32),
            jax.ShapeDtypeStruct((TC_ROWS, N_ATTRIBUTES), jnp.float32),
        ],
    )(sender_input)


def kernel(sender_input, labels):
    masked_lo, mask_lo = _tc_run(sender_input)
    masked_hi, mask_hi = _sc_run(sender_input)
    masked_input = jnp.concatenate([masked_lo, masked_hi], axis=0)
    mask = jnp.concatenate([mask_lo, mask_hi], axis=0)
    return masked_input, mask


# speedup vs baseline: 1.1561x; 1.1561x over previous
import functools

import jax
import jax.numpy as jnp
from jax import lax
from jax.experimental import pallas as pl
from jax.experimental.pallas import tpu as pltpu
from jax.experimental.pallas import tpu_sc as plsc

BATCH = 4096
WIDTH = 5000
N_ATTRIBUTES = 50
N_VALUES = 100
CURRICULUM_LEVEL = 25
HALF = 2560
LANES = 16
CHUNK = 16
ROWS = 256


def _threefry_mantissa(counts_lo):
    u32 = jnp.uint32
    ks0 = u32(0)
    ks1 = u32(42)
    ks2 = u32(0x1BD11BDA ^ 42)

    def rotl(x, r):
        return lax.shift_left(x, u32(r)) | lax.shift_right_logical(
            x, u32(32 - r))

    def four_rounds(x0, x1, rots):
        for r in rots:
            x0 = x0 + x1
            x1 = rotl(x1, r) ^ x0
        return x0, x1

    R0 = (13, 15, 26, 6)
    R1 = (17, 29, 16, 24)
    x0 = jnp.zeros_like(counts_lo) + ks0
    x1 = counts_lo + ks1
    x0, x1 = four_rounds(x0, x1, R0)
    x0 = x0 + ks1
    x1 = x1 + ks2 + u32(1)
    x0, x1 = four_rounds(x0, x1, R1)
    x0 = x0 + ks2
    x1 = x1 + ks0 + u32(2)
    x0, x1 = four_rounds(x0, x1, R0)
    x0 = x0 + ks0
    x1 = x1 + ks1 + u32(3)
    x0, x1 = four_rounds(x0, x1, R1)
    x0 = x0 + ks1
    x1 = x1 + ks2 + u32(4)
    x0, x1 = four_rounds(x0, x1, R0)
    x0 = x0 + ks2
    x1 = x1 + ks0 + u32(5)
    return lax.shift_right_logical(x0 ^ x1, u32(9))


def _n_revealed_lanes(row0):
    rows25 = (row0 + lax.iota(jnp.int32, LANES)) * CURRICULUM_LEVEL

    def step(_, carry):
        best_m, best_j, jv = carry
        counts = (rows25 + jv).astype(jnp.uint32)
        m = _threefry_mantissa(counts).astype(jnp.int32)
        take = m > best_m
        return (jnp.where(take, m, best_m), jnp.where(take, jv, best_j),
                jv + 1)

    init = (jnp.full((LANES,), -1, jnp.int32), jnp.zeros((LANES,), jnp.int32),
            jnp.zeros((LANES,), jnp.int32))
    best_m, best_j, _ = lax.fori_loop(0, CURRICULUM_LEVEL, step, init)
    return best_j + 1


def _sc_mask_body(mask_hbm, maskbuf):
    info = plsc.get_sparse_core_info()
    nc = info.num_cores
    wid = lax.axis_index("s") * nc + lax.axis_index("c")
    rows_per_worker = BATCH // (nc * info.num_subcores)
    nchunks = rows_per_worker // CHUNK
    base = wid * rows_per_worker
    zeros16 = jnp.zeros((LANES,), jnp.float32)
    lane = lax.iota(jnp.int32, LANES)

    def chunk_body(c, _):
        r0 = base + c * CHUNK
        n_rev = _n_revealed_lanes(r0)
        for r in range(CHUNK):
            nr = n_rev[r]
            for off in (0, 16, 32, 34):
                mval = jnp.where(lane + jnp.int32(off) < nr,
                                 jnp.float32(1.0), zeros16)
                maskbuf[r, pl.ds(off, LANES)] = mval
        pltpu.sync_copy(maskbuf, mask_hbm.at[pl.ds(r0, CHUNK), :])
        return 0

    lax.fori_loop(0, nchunks, chunk_body, 0)


def _sc_mask():
    mesh = plsc.VectorSubcoreMesh(core_axis_name="c", subcore_axis_name="s")
    run = functools.partial(
        pl.kernel,
        out_type=jax.ShapeDtypeStruct((BATCH, N_ATTRIBUTES), jnp.float32),
        mesh=mesh,
        scratch_types=[pltpu.VMEM((CHUNK, N_ATTRIBUTES), jnp.float32)],
    )(_sc_mask_body)
    return run()


def _n_revealed_block(base_row, rows):
    cl = CURRICULUM_LEVEL
    r = lax.broadcasted_iota(jnp.uint32, (rows, cl), 0)
    j = lax.broadcasted_iota(jnp.uint32, (rows, cl), 1)
    counts = (base_row.astype(jnp.uint32) + r) * jnp.uint32(cl) + j
    m = _threefry_mantissa(counts).astype(jnp.int32)
    row_max = jnp.max(m, axis=1, keepdims=True)
    ji = lax.broadcasted_iota(jnp.int32, (rows, cl), 1)
    win = jnp.min(jnp.where(m == row_max, ji, jnp.int32(cl)), axis=1,
                  keepdims=True)
    return win + 1


def _tc_body(x_ref, masked_ref):
    i = pl.program_id(0)
    n_rev = _n_revealed_block(i * ROWS, ROWS)
    c = lax.broadcasted_iota(jnp.int32, (ROWS, HALF), 1)
    masked_ref[:, 0:HALF] = jnp.where(c < n_rev * N_VALUES, x_ref[...], 0.0)
    masked_ref[:, HALF:] = jnp.zeros((ROWS, WIDTH - HALF), jnp.float32)


def _tc_masked(sender_input):
    grid = BATCH // ROWS
    return pl.pallas_call(
        _tc_body,
        grid=(grid,),
        in_specs=[pl.BlockSpec((ROWS, HALF), lambda i: (i, 0))],
        out_specs=pl.BlockSpec((ROWS, WIDTH), lambda i: (i, 0)),
        out_shape=jax.ShapeDtypeStruct((BATCH, WIDTH), jnp.float32),
    )(sender_input)


def kernel(sender_input, labels):
    return _tc_masked(sender_input), _sc_mask()


# speedup vs baseline: 1.1707x; 1.0126x over previous
import functools

import jax
import jax.numpy as jnp
from jax import lax
from jax.experimental import pallas as pl
from jax.experimental.pallas import tpu as pltpu
from jax.experimental.pallas import tpu_sc as plsc

BATCH = 4096
WIDTH = 5000
N_ATTRIBUTES = 50
N_VALUES = 100
CURRICULUM_LEVEL = 25
HALF = 2560
LANES = 16
CHUNK = 16
ROWS = 512


def _threefry_mantissa(counts_lo):
    u32 = jnp.uint32
    ks0 = u32(0)
    ks1 = u32(42)
    ks2 = u32(0x1BD11BDA ^ 42)

    def rotl(x, r):
        return lax.shift_left(x, u32(r)) | lax.shift_right_logical(
            x, u32(32 - r))

    def four_rounds(x0, x1, rots):
        for r in rots:
            x0 = x0 + x1
            x1 = rotl(x1, r) ^ x0
        return x0, x1

    R0 = (13, 15, 26, 6)
    R1 = (17, 29, 16, 24)
    x0 = jnp.zeros_like(counts_lo) + ks0
    x1 = counts_lo + ks1
    x0, x1 = four_rounds(x0, x1, R0)
    x0 = x0 + ks1
    x1 = x1 + ks2 + u32(1)
    x0, x1 = four_rounds(x0, x1, R1)
    x0 = x0 + ks2
    x1 = x1 + ks0 + u32(2)
    x0, x1 = four_rounds(x0, x1, R0)
    x0 = x0 + ks0
    x1 = x1 + ks1 + u32(3)
    x0, x1 = four_rounds(x0, x1, R1)
    x0 = x0 + ks1
    x1 = x1 + ks2 + u32(4)
    x0, x1 = four_rounds(x0, x1, R0)
    x0 = x0 + ks2
    x1 = x1 + ks0 + u32(5)
    return lax.shift_right_logical(x0 ^ x1, u32(9))


def _n_revealed_lanes(row0):
    rows25 = (row0 + lax.iota(jnp.int32, LANES)) * CURRICULUM_LEVEL

    def step(_, carry):
        best_m, best_j, jv = carry
        counts = (rows25 + jv).astype(jnp.uint32)
        m = _threefry_mantissa(counts).astype(jnp.int32)
        take = m > best_m
        return (jnp.where(take, m, best_m), jnp.where(take, jv, best_j),
                jv + 1)

    init = (jnp.full((LANES,), -1, jnp.int32), jnp.zeros((LANES,), jnp.int32),
            jnp.zeros((LANES,), jnp.int32))
    best_m, best_j, _ = lax.fori_loop(0, CURRICULUM_LEVEL, step, init)
    return best_j + 1


def _sc_mask_body(mask_hbm, maskbuf):
    info = plsc.get_sparse_core_info()
    nc = info.num_cores
    wid = lax.axis_index("s") * nc + lax.axis_index("c")
    rows_per_worker = BATCH // (nc * info.num_subcores)
    nchunks = rows_per_worker // CHUNK
    base = wid * rows_per_worker
    zeros16 = jnp.zeros((LANES,), jnp.float32)
    lane = lax.iota(jnp.int32, LANES)

    def chunk_body(c, _):
        r0 = base + c * CHUNK
        n_rev = _n_revealed_lanes(r0)
        for r in range(CHUNK):
            nr = n_rev[r]
            for off in (0, 16, 32, 34):
                mval = jnp.where(lane + jnp.int32(off) < nr,
                                 jnp.float32(1.0), zeros16)
                maskbuf[r, pl.ds(off, LANES)] = mval
        pltpu.sync_copy(maskbuf, mask_hbm.at[pl.ds(r0, CHUNK), :])
        return 0

    lax.fori_loop(0, nchunks, chunk_body, 0)


def _sc_mask():
    mesh = plsc.VectorSubcoreMesh(core_axis_name="c", subcore_axis_name="s")
    run = functools.partial(
        pl.kernel,
        out_type=jax.ShapeDtypeStruct((BATCH, N_ATTRIBUTES), jnp.float32),
        mesh=mesh,
        scratch_types=[pltpu.VMEM((CHUNK, N_ATTRIBUTES), jnp.float32)],
    )(_sc_mask_body)
    return run()


def _n_revealed_block(base_row, rows):
    cl = CURRICULUM_LEVEL
    r = lax.broadcasted_iota(jnp.uint32, (rows, cl), 0)
    j = lax.broadcasted_iota(jnp.uint32, (rows, cl), 1)
    counts = (base_row.astype(jnp.uint32) + r) * jnp.uint32(cl) + j
    m = _threefry_mantissa(counts).astype(jnp.int32)
    row_max = jnp.max(m, axis=1, keepdims=True)
    ji = lax.broadcasted_iota(jnp.int32, (rows, cl), 1)
    win = jnp.min(jnp.where(m == row_max, ji, jnp.int32(cl)), axis=1,
                  keepdims=True)
    return win + 1


def _tc_body(x_ref, masked_ref):
    i = pl.program_id(0)
    n_rev = _n_revealed_block(i * ROWS, ROWS)
    c = lax.broadcasted_iota(jnp.int32, (ROWS, HALF), 1)
    masked_ref[:, 0:HALF] = jnp.where(c < n_rev * N_VALUES, x_ref[...], 0.0)
    masked_ref[:, HALF:] = jnp.zeros((ROWS, WIDTH - HALF), jnp.float32)


def _tc_masked(sender_input):
    grid = BATCH // ROWS
    return pl.pallas_call(
        _tc_body,
        grid=(grid,),
        in_specs=[pl.BlockSpec((ROWS, HALF), lambda i: (i, 0))],
        out_specs=pl.BlockSpec((ROWS, WIDTH), lambda i: (i, 0)),
        out_shape=jax.ShapeDtypeStruct((BATCH, WIDTH), jnp.float32),
    )(sender_input)


def kernel(sender_input, labels):
    return _tc_masked(sender_input), _sc_mask()


# speedup vs baseline: 1.1789x; 1.0070x over previous
import functools

import jax
import jax.numpy as jnp
from jax import lax
from jax.experimental import pallas as pl
from jax.experimental.pallas import tpu as pltpu
from jax.experimental.pallas import tpu_sc as plsc

BATCH = 4096
WIDTH = 5000
N_ATTRIBUTES = 50
N_VALUES = 100
CURRICULUM_LEVEL = 25
HALF = 2560
LANES = 16
CHUNK = 16
ROWS = 512


def _threefry_mantissa(counts_lo):
    u32 = jnp.uint32
    ks0 = u32(0)
    ks1 = u32(42)
    ks2 = u32(0x1BD11BDA ^ 42)

    def rotl(x, r):
        return lax.shift_left(x, u32(r)) | lax.shift_right_logical(
            x, u32(32 - r))

    def four_rounds(x0, x1, rots):
        for r in rots:
            x0 = x0 + x1
            x1 = rotl(x1, r) ^ x0
        return x0, x1

    R0 = (13, 15, 26, 6)
    R1 = (17, 29, 16, 24)
    x0 = jnp.zeros_like(counts_lo) + ks0
    x1 = counts_lo + ks1
    x0, x1 = four_rounds(x0, x1, R0)
    x0 = x0 + ks1
    x1 = x1 + ks2 + u32(1)
    x0, x1 = four_rounds(x0, x1, R1)
    x0 = x0 + ks2
    x1 = x1 + ks0 + u32(2)
    x0, x1 = four_rounds(x0, x1, R0)
    x0 = x0 + ks0
    x1 = x1 + ks1 + u32(3)
    x0, x1 = four_rounds(x0, x1, R1)
    x0 = x0 + ks1
    x1 = x1 + ks2 + u32(4)
    x0, x1 = four_rounds(x0, x1, R0)
    x0 = x0 + ks2
    x1 = x1 + ks0 + u32(5)
    return lax.shift_right_logical(x0 ^ x1, u32(9))


def _n_revealed_lanes(row0):
    rows25 = (row0 + lax.iota(jnp.int32, LANES)) * CURRICULUM_LEVEL

    def step(_, carry):
        best_m, best_j, jv = carry
        counts = (rows25 + jv).astype(jnp.uint32)
        m = _threefry_mantissa(counts).astype(jnp.int32)
        take = m > best_m
        return (jnp.where(take, m, best_m), jnp.where(take, jv, best_j),
                jv + 1)

    init = (jnp.full((LANES,), -1, jnp.int32), jnp.zeros((LANES,), jnp.int32),
            jnp.zeros((LANES,), jnp.int32))
    best_m, best_j, _ = lax.fori_loop(0, CURRICULUM_LEVEL, step, init)
    return best_j + 1


def _sc_mask_body(masked_hbm, mask_hbm, zbuf, maskbuf):
    info = plsc.get_sparse_core_info()
    nc = info.num_cores
    wid = lax.axis_index("s") * nc + lax.axis_index("c")
    rows_per_worker = BATCH // (nc * info.num_subcores)
    nchunks = rows_per_worker // CHUNK
    base = wid * rows_per_worker
    zeros16 = jnp.zeros((LANES,), jnp.float32)
    lane = lax.iota(jnp.int32, LANES)
    ztail = WIDTH - HALF

    nz8 = ztail // 128
    for r in range(CHUNK):
        def zinit(t, _, r=r):
            s0 = t * 128
            for k in range(8):
                zbuf[r, pl.ds(s0 + k * 16, LANES)] = zeros16
            return 0
        lax.fori_loop(0, nz8, zinit, 0)
        zbuf[r, pl.ds(ztail - LANES, LANES)] = zeros16

    def chunk_body(c, _):
        r0 = base + c * CHUNK
        n_rev = _n_revealed_lanes(r0)
        for r in range(CHUNK):
            nr = n_rev[r]
            for off in (0, 16, 32, 34):
                mval = jnp.where(lane + jnp.int32(off) < nr,
                                 jnp.float32(1.0), zeros16)
                maskbuf[r, pl.ds(off, LANES)] = mval
        pltpu.sync_copy(maskbuf, mask_hbm.at[pl.ds(r0, CHUNK), :])
        pltpu.sync_copy(zbuf,
                        masked_hbm.at[pl.ds(r0, CHUNK), pl.ds(HALF, ztail)])
        return 0

    lax.fori_loop(0, nchunks, chunk_body, 0)


def _sc_mask():
    mesh = plsc.VectorSubcoreMesh(core_axis_name="c", subcore_axis_name="s")
    run = functools.partial(
        pl.kernel,
        out_type=[
            jax.ShapeDtypeStruct((BATCH, WIDTH), jnp.float32),
            jax.ShapeDtypeStruct((BATCH, N_ATTRIBUTES), jnp.float32),
        ],
        mesh=mesh,
        scratch_types=[
            pltpu.VMEM((CHUNK, WIDTH - HALF), jnp.float32),
            pltpu.VMEM((CHUNK, N_ATTRIBUTES), jnp.float32),
        ],
    )(_sc_mask_body)
    return run()


def _n_revealed_block(base_row, rows):
    cl = CURRICULUM_LEVEL
    r = lax.broadcasted_iota(jnp.uint32, (rows, cl), 0)
    j = lax.broadcasted_iota(jnp.uint32, (rows, cl), 1)
    counts = (base_row.astype(jnp.uint32) + r) * jnp.uint32(cl) + j
    m = _threefry_mantissa(counts).astype(jnp.int32)
    row_max = jnp.max(m, axis=1, keepdims=True)
    ji = lax.broadcasted_iota(jnp.int32, (rows, cl), 1)
    win = jnp.min(jnp.where(m == row_max, ji, jnp.int32(cl)), axis=1,
                  keepdims=True)
    return win + 1


def _tc_body(x_ref, alias_ref, masked_ref):
    del alias_ref
    i = pl.program_id(0)
    n_rev = _n_revealed_block(i * ROWS, ROWS)
    c = lax.broadcasted_iota(jnp.int32, (ROWS, HALF), 1)
    masked_ref[...] = jnp.where(c < n_rev * N_VALUES, x_ref[...], 0.0)


def _tc_masked(sender_input, masked_init):
    grid = BATCH // ROWS
    return pl.pallas_call(
        _tc_body,
        grid=(grid,),
        in_specs=[
            pl.BlockSpec((ROWS, HALF), lambda i: (i, 0)),
            pl.BlockSpec(memory_space=pl.ANY),
        ],
        out_specs=pl.BlockSpec((ROWS, HALF), lambda i: (i, 0)),
        out_shape=jax.ShapeDtypeStruct((BATCH, WIDTH), jnp.float32),
        input_output_aliases={1: 0},
    )(sender_input, masked_init)


def kernel(sender_input, labels):
    masked_init, mask = _sc_mask()
    return _tc_masked(sender_input, masked_init), mask


# speedup vs baseline: 1.1804x; 1.0013x over previous
import functools

import jax
import jax.numpy as jnp
from jax import lax
from jax.experimental import pallas as pl
from jax.experimental.pallas import tpu as pltpu
from jax.experimental.pallas import tpu_sc as plsc

BATCH = 4096
WIDTH = 5000
N_ATTRIBUTES = 50
N_VALUES = 100
CURRICULUM_LEVEL = 25
HALF = 2560
LANES = 16
CHUNK = 16
ROWS = 512


def _threefry_mantissa(counts_lo):
    u32 = jnp.uint32
    ks0 = u32(0)
    ks1 = u32(42)
    ks2 = u32(0x1BD11BDA ^ 42)

    def rotl(x, r):
        return lax.shift_left(x, u32(r)) | lax.shift_right_logical(
            x, u32(32 - r))

    def four_rounds(x0, x1, rots):
        for r in rots:
            x0 = x0 + x1
            x1 = rotl(x1, r) ^ x0
        return x0, x1

    R0 = (13, 15, 26, 6)
    R1 = (17, 29, 16, 24)
    x0 = jnp.zeros_like(counts_lo) + ks0
    x1 = counts_lo + ks1
    x0, x1 = four_rounds(x0, x1, R0)
    x0 = x0 + ks1
    x1 = x1 + ks2 + u32(1)
    x0, x1 = four_rounds(x0, x1, R1)
    x0 = x0 + ks2
    x1 = x1 + ks0 + u32(2)
    x0, x1 = four_rounds(x0, x1, R0)
    x0 = x0 + ks0
    x1 = x1 + ks1 + u32(3)
    x0, x1 = four_rounds(x0, x1, R1)
    x0 = x0 + ks1
    x1 = x1 + ks2 + u32(4)
    x0, x1 = four_rounds(x0, x1, R0)
    x0 = x0 + ks2
    x1 = x1 + ks0 + u32(5)
    return lax.shift_right_logical(x0 ^ x1, u32(9))


def _n_revealed_lanes(row0):
    rows25 = (row0 + lax.iota(jnp.int32, LANES)) * CURRICULUM_LEVEL

    def step(_, carry):
        best_m, best_j, jv = carry
        counts = (rows25 + jv).astype(jnp.uint32)
        m = _threefry_mantissa(counts).astype(jnp.int32)
        take = m > best_m
        return (jnp.where(take, m, best_m), jnp.where(take, jv, best_j),
                jv + 1)

    init = (jnp.full((LANES,), -1, jnp.int32), jnp.zeros((LANES,), jnp.int32),
            jnp.zeros((LANES,), jnp.int32))
    best_m, best_j, _ = lax.fori_loop(0, CURRICULUM_LEVEL, step, init)
    return best_j + 1


def _sc_zeros_and_mask_body(masked_hbm, mask_hbm, zbuf, maskbuf):
    info = plsc.get_sparse_core_info()
    nc = info.num_cores
    wid = lax.axis_index("s") * nc + lax.axis_index("c")
    rows_per_worker = BATCH // (nc * info.num_subcores)
    nchunks = rows_per_worker // CHUNK
    base = wid * rows_per_worker
    zeros16 = jnp.zeros((LANES,), jnp.float32)
    lane = lax.iota(jnp.int32, LANES)
    ztail = WIDTH - HALF

    nz8 = ztail // 128
    for r in range(CHUNK):
        def zinit(t, _, r=r):
            s0 = t * 128
            for k in range(8):
                zbuf[r, pl.ds(s0 + k * 16, LANES)] = zeros16
            return 0
        lax.fori_loop(0, nz8, zinit, 0)
        zbuf[r, pl.ds(ztail - LANES, LANES)] = zeros16

    def chunk_body(c, _):
        r0 = base + c * CHUNK
        n_rev = _n_revealed_lanes(r0)
        for r in range(CHUNK):
            nr = n_rev[r]
            for off in (0, 16, 32, 34):
                mval = jnp.where(lane + jnp.int32(off) < nr,
                                 jnp.float32(1.0), zeros16)
                maskbuf[r, pl.ds(off, LANES)] = mval
        pltpu.sync_copy(maskbuf, mask_hbm.at[pl.ds(r0, CHUNK), :])
        pltpu.sync_copy(zbuf,
                        masked_hbm.at[pl.ds(r0, CHUNK), pl.ds(HALF, ztail)])
        return 0

    lax.fori_loop(0, nchunks, chunk_body, 0)


def _sc_zeros_and_mask():
    mesh = plsc.VectorSubcoreMesh(core_axis_name="c", subcore_axis_name="s")
    run = functools.partial(
        pl.kernel,
        out_type=[
            jax.ShapeDtypeStruct((BATCH, WIDTH), jnp.float32),
            jax.ShapeDtypeStruct((BATCH, N_ATTRIBUTES), jnp.float32),
        ],
        mesh=mesh,
        scratch_types=[
            pltpu.VMEM((CHUNK, WIDTH - HALF), jnp.float32),
            pltpu.VMEM((CHUNK, N_ATTRIBUTES), jnp.float32),
        ],
    )(_sc_zeros_and_mask_body)
    return run()


def _n_revealed_block(base_row, rows):
    cl = CURRICULUM_LEVEL
    r = lax.broadcasted_iota(jnp.uint32, (rows, cl), 0)
    j = lax.broadcasted_iota(jnp.uint32, (rows, cl), 1)
    counts = (base_row.astype(jnp.uint32) + r) * jnp.uint32(cl) + j
    m = _threefry_mantissa(counts).astype(jnp.int32)
    row_max = jnp.max(m, axis=1, keepdims=True)
    ji = lax.broadcasted_iota(jnp.int32, (rows, cl), 1)
    win = jnp.min(jnp.where(m == row_max, ji, jnp.int32(cl)), axis=1,
                  keepdims=True)
    return win + 1


def _tc_body(x_ref, alias_ref, masked_ref):
    del alias_ref
    i = pl.program_id(0)
    n_rev = _n_revealed_block(i * ROWS, ROWS)
    c = lax.broadcasted_iota(jnp.int32, (ROWS, HALF), 1)
    masked_ref[...] = jnp.where(c < n_rev * N_VALUES, x_ref[...], 0.0)


def _tc_masked(sender_input, masked_init):
    grid = BATCH // ROWS
    return pl.pallas_call(
        _tc_body,
        grid=(grid,),
        in_specs=[
            pl.BlockSpec((ROWS, HALF), lambda i: (i, 0)),
            pl.BlockSpec(memory_space=pl.ANY),
        ],
        out_specs=pl.BlockSpec((ROWS, HALF), lambda i: (i, 0)),
        out_shape=jax.ShapeDtypeStruct((BATCH, WIDTH), jnp.float32),
        input_output_aliases={1: 0},
    )(sender_input, masked_init)


def kernel(sender_input, labels):
    masked_init, mask = _sc_zeros_and_mask()
    return _tc_masked(sender_input, masked_init), mask
